# baseline (speedup 1.0000x reference)
import functools

import jax
import jax.numpy as jnp
from jax import lax
from jax.experimental import pallas as pl
from jax.experimental.pallas import tpu as pltpu
from jax.experimental.pallas import tpu_sc as plsc

_NUM_EMB = 1000000
_HALF = 32
_BATCH = 16384
_FIELDS = 26
_BF = _BATCH * _FIELDS
_NC = 2
_NS = 16
_NW = _NC * _NS
_PER_W = _BF // _NW
_GRP = 128
_NG = _PER_W // _GRP
_BLK = 512
_GPB = _BLK // _GRP
_NBLK = _PER_W // _BLK

_VT_FULL = _NUM_EMB // _GRP
_VT_ROUNDS = -(-_VT_FULL // _NW)
_TAIL = _NUM_EMB - _VT_FULL * _GRP
_OROWS = _NUM_EMB * _HALF // 128


_RB = 8192
_RGRID = -(-_NUM_EMB // _RB)


def _relayout_tc(t1t, t2t):
    def body(in1_ref, in2_ref, o1_ref, o2_ref):
        for in_ref, o_ref in ((in1_ref, o1_ref), (in2_ref, o2_ref)):
            t = in_ref[...].T.reshape(_RB // 4, 4, _HALF)
            o_ref[...] = jnp.concatenate(
                [t[:, v, :] for v in range(4)], axis=1)

    otype = jax.ShapeDtypeStruct((_OROWS, 128), jnp.float32)
    ispec = pl.BlockSpec((_HALF, _RB), lambda i: (0, i))
    ospec = pl.BlockSpec((_RB // 4, 128), lambda i: (i, 0))
    return pl.pallas_call(
        body,
        grid=(_RGRID,),
        in_specs=[ispec, ispec],
        out_specs=[ospec, ospec],
        out_shape=(otype, otype),
    )(t1t, t2t)


_BPW = _BATCH // _NW
_BC = 64
_NCHK = _BPW // _BC
_FPC = _BC * _FIELDS
_SPC = _FPC // _GRP


def _dual_gather(x_grp, table1, table2):
    mesh = plsc.VectorSubcoreMesh(core_axis_name="c", subcore_axis_name="s")

    @functools.partial(
        pl.kernel,
        mesh=mesh,
        compiler_params=pltpu.CompilerParams(
            use_tc_tiling_on_sc=False, needs_layout_passes=False),
        out_type=jax.ShapeDtypeStruct((_FIELDS, 2 * _HALF, _BATCH),
                                      jnp.float32),
        scratch_types=[
            pltpu.VMEM((_NG, _GRP), jnp.int32),
            pltpu.VMEM((_FPC, _HALF), jnp.float32),
            pltpu.VMEM((_FPC, _HALF), jnp.float32),
            pltpu.VMEM((4, _HALF, _BC), jnp.float32),
            pltpu.SemaphoreType.DMA,
            pltpu.SemaphoreType.DMA,
            pltpu.SemaphoreType.DMA,
            pltpu.SemaphoreType.DMA,
            pltpu.SemaphoreType.DMA,
            pltpu.SemaphoreType.DMA,
        ],
    )
    def k(x_hbm, t1_hbm, t2_hbm, out_hbm, idx_v, rows1_v, rows2_v, trans_v,
          gsem1, gsem2, ws0, ws1, ws2, ws3):
        wid = lax.axis_index("s") * _NC + lax.axis_index("c")
        b0 = wid * _BPW
        iota26 = lax.iota(jnp.int32, 16) * _FIELDS
        pltpu.sync_copy(x_hbm.at[wid], idx_v)

        def fire(j, tab_hbm, rows_v, gsem):
            for s in range(_SPC):
                pltpu.async_copy(
                    tab_hbm.at[idx_v.at[j * _SPC + s]],
                    rows_v.at[pl.ds(s * _GRP, _GRP)], gsem)

        def drain(j, tab_hbm, rows_v, gsem):
            for s in range(_SPC):
                pltpu.make_async_copy(
                    tab_hbm.at[idx_v.at[j * _SPC + s]],
                    rows_v.at[pl.ds(s * _GRP, _GRP)], gsem).wait()

        def plane_sweep(j, rows_v, dbase, buf_a, buf_b, sem_a, sem_b):
            cbase = b0 + j * _BC

            def wr_desc(f, buf, sem):
                return pltpu.make_async_copy(
                    trans_v.at[buf],
                    out_hbm.at[f, pl.ds(dbase, _HALF), pl.ds(cbase, _BC)],
                    sem)

            def q_body(q, carry):
                f0 = 2 * q

                @pl.when(q > 0)
                def _():
                    wr_desc(f0 - 2, buf_a, sem_a).wait()
                    wr_desc(f0 - 1, buf_b, sem_b).wait()

                for foff, buf, sem in ((0, buf_a, sem_a), (1, buf_b, sem_b)):
                    f = f0 + foff
                    rvecs = [iota26 + (b16 * (16 * _FIELDS) + f)
                             for b16 in range(_BC // 16)]
                    for d in range(_HALF):
                        cols = jnp.full((16,), d, jnp.int32)
                        for b16 in range(_BC // 16):
                            trans_v[buf, d, pl.ds(b16 * 16, 16)] = (
                                plsc.load_gather(rows_v, [rvecs[b16], cols]))
                    wr_desc(f, buf, sem).start()
                return carry

            lax.fori_loop(0, _FIELDS // 2, q_body, 0)
            wr_desc(_FIELDS - 2, buf_a, sem_a).wait()
            wr_desc(_FIELDS - 1, buf_b, sem_b).wait()

        fire(0, t1_hbm, rows1_v, gsem1)
        fire(0, t2_hbm, rows2_v, gsem2)

        def chunk_body(j, carry):
            drain(j, t1_hbm, rows1_v, gsem1)
            plane_sweep(j, rows1_v, 0, 0, 1, ws0, ws1)

            @pl.when(j + 1 < _NCHK)
            def _():
                fire(j + 1, t1_hbm, rows1_v, gsem1)

            drain(j, t2_hbm, rows2_v, gsem2)
            plane_sweep(j, rows2_v, _HALF, 2, 3, ws2, ws3)

            @pl.when(j + 1 < _NCHK)
            def _():
                fire(j + 1, t2_hbm, rows2_v, gsem2)
            return carry

        lax.fori_loop(0, _NCHK, chunk_body, 0)

    return k(x_grp, table1, table2)


def kernel(x, table1, table2):
    t1t = jnp.swapaxes(table1, 0, 1)
    t2t = jnp.swapaxes(table2, 0, 1)
    r1, r2 = _relayout_tc(t1t, t2t)
    x_grp = x.reshape(_NW, _NG, _GRP).astype(jnp.int32)
    out = _dual_gather(x_grp, r1.reshape(_NUM_EMB, _HALF),
                       r2.reshape(_NUM_EMB, _HALF))
    return jnp.transpose(out, (2, 0, 1))


# speedup vs baseline: 1.3864x; 1.3864x over previous
import functools

import jax
import jax.numpy as jnp
from jax import lax
from jax.experimental import pallas as pl
from jax.experimental.pallas import tpu as pltpu
from jax.experimental.pallas import tpu_sc as plsc

_NUM_EMB = 1000000
_HALF = 32
_BATCH = 16384
_FIELDS = 26
_BF = _BATCH * _FIELDS
_NC = 2
_NS = 16
_NW = _NC * _NS
_PER_W = _BF // _NW
_GRP = 128
_NG = _PER_W // _GRP
_BLK = 512
_GPB = _BLK // _GRP
_NBLK = _PER_W // _BLK

_VT_FULL = _NUM_EMB // _GRP
_VT_ROUNDS = -(-_VT_FULL // _NW)
_TAIL = _NUM_EMB - _VT_FULL * _GRP
_OROWS = _NUM_EMB * _HALF // 128


_RB = 8192
_RGRID = -(-_NUM_EMB // _RB)


def _relayout_tc(t1t, t2t):
    def body(in1_ref, in2_ref, o1_ref, o2_ref):
        for in_ref, o_ref in ((in1_ref, o1_ref), (in2_ref, o2_ref)):
            t = in_ref[...].T.reshape(_RB // 4, 4, _HALF)
            o_ref[...] = jnp.concatenate(
                [t[:, v, :] for v in range(4)], axis=1)

    otype = jax.ShapeDtypeStruct((_OROWS, 128), jnp.float32)
    ispec = pl.BlockSpec((_HALF, _RB), lambda i: (0, i))
    ospec = pl.BlockSpec((_RB // 4, 128), lambda i: (i, 0))
    return pl.pallas_call(
        body,
        grid=(_RGRID,),
        in_specs=[ispec, ispec],
        out_specs=[ospec, ospec],
        out_shape=(otype, otype),
    )(t1t, t2t)


def _dual_gather(x_grp, table1, table2):
    mesh = plsc.VectorSubcoreMesh(core_axis_name="c", subcore_axis_name="s")

    @functools.partial(
        pl.kernel,
        mesh=mesh,
        compiler_params=pltpu.CompilerParams(use_tc_tiling_on_sc=False),
        out_type=jax.ShapeDtypeStruct((_BF, 2 * _HALF), jnp.float32),
        scratch_types=[
            pltpu.VMEM((_NG, _GRP), jnp.int32),
            pltpu.VMEM((2, _BLK, _HALF), jnp.float32),
            pltpu.VMEM((2, _BLK, _HALF), jnp.float32),
            pltpu.SemaphoreType.DMA,
            pltpu.SemaphoreType.DMA,
        ],
    )
    def k(x_hbm, t1_hbm, t2_hbm, out_hbm, idx_v, rows1_v, rows2_v,
          gsem, wsem):
        wid = lax.axis_index("s") * _NC + lax.axis_index("c")
        base = wid * _PER_W
        pltpu.sync_copy(x_hbm.at[wid], idx_v)

        def outer(i, carry):
            for b in range(2):
                j = 2 * i + b
                ghs = []
                for g in range(_GPB):
                    row = j * _GPB + g
                    ghs.append(pltpu.async_copy(
                        t1_hbm.at[idx_v.at[row]],
                        rows1_v.at[b].at[pl.ds(g * _GRP, _GRP)], gsem))
                    ghs.append(pltpu.async_copy(
                        t2_hbm.at[idx_v.at[row]],
                        rows2_v.at[b].at[pl.ds(g * _GRP, _GRP)], gsem))

                pb = 1 - b
                pbase = base + (j - 1) * _BLK

                @pl.when(j > 0)
                def _fire_writes():
                    pltpu.async_copy(
                        rows1_v.at[pb],
                        out_hbm.at[pl.ds(pbase, _BLK), pl.ds(0, _HALF)],
                        wsem)
                    pltpu.async_copy(
                        rows2_v.at[pb],
                        out_hbm.at[pl.ds(pbase, _BLK), pl.ds(_HALF, _HALF)],
                        wsem)

                for h in ghs:
                    h.wait()

                @pl.when(j > 0)
                def _wait_writes():
                    pltpu.make_async_copy(
                        rows1_v.at[pb],
                        out_hbm.at[pl.ds(pbase, _BLK), pl.ds(0, _HALF)],
                        wsem).wait()
                    pltpu.make_async_copy(
                        rows2_v.at[pb],
                        out_hbm.at[pl.ds(pbase, _BLK), pl.ds(_HALF, _HALF)],
                        wsem).wait()
            return carry

        lax.fori_loop(0, _NBLK // 2, outer, 0)

        lbase = base + (_NBLK - 1) * _BLK
        pltpu.sync_copy(
            rows1_v.at[1], out_hbm.at[pl.ds(lbase, _BLK), pl.ds(0, _HALF)])
        pltpu.sync_copy(
            rows2_v.at[1],
            out_hbm.at[pl.ds(lbase, _BLK), pl.ds(_HALF, _HALF)])

    return k(x_grp, table1, table2)


def kernel(x, table1, table2):
    t1t = jnp.swapaxes(table1, 0, 1)
    t2t = jnp.swapaxes(table2, 0, 1)
    r1, r2 = _relayout_tc(t1t, t2t)
    x_grp = x.reshape(_NW, _NG, _GRP).astype(jnp.int32)
    out = _dual_gather(x_grp, r1.reshape(_NUM_EMB, _HALF),
                       r2.reshape(_NUM_EMB, _HALF))
    return out.reshape(_BATCH, _FIELDS, 2 * _HALF)


# speedup vs baseline: 1.3902x; 1.0028x over previous
import functools

import jax
import jax.numpy as jnp
from jax import lax
from jax.experimental import pallas as pl
from jax.experimental.pallas import tpu as pltpu
from jax.experimental.pallas import tpu_sc as plsc

_NUM_EMB = 1000000
_HALF = 32
_BATCH = 16384
_FIELDS = 26
_BF = _BATCH * _FIELDS
_NC = 2
_NS = 16
_NW = _NC * _NS
_PER_W = _BF // _NW
_GRP = 128
_NG = _PER_W // _GRP
_BLK = 512
_GPB = _BLK // _GRP
_NBLK = _PER_W // _BLK

_VT_FULL = _NUM_EMB // _GRP
_VT_ROUNDS = -(-_VT_FULL // _NW)
_TAIL = _NUM_EMB - _VT_FULL * _GRP
_OROWS = _NUM_EMB * _HALF // 128


_RB = 16384
_RGRID = -(-_NUM_EMB // _RB)


def _relayout_tc(t1t, t2t):
    def body(in1_ref, in2_ref, o1_ref, o2_ref):
        for in_ref, o_ref in ((in1_ref, o1_ref), (in2_ref, o2_ref)):
            t = in_ref[...].T.reshape(_RB // 4, 4, _HALF)
            o_ref[...] = jnp.concatenate(
                [t[:, v, :] for v in range(4)], axis=1)

    otype = jax.ShapeDtypeStruct((_OROWS, 128), jnp.float32)
    ispec = pl.BlockSpec((_HALF, _RB), lambda i: (0, i))
    ospec = pl.BlockSpec((_RB // 4, 128), lambda i: (i, 0))
    return pl.pallas_call(
        body,
        grid=(_RGRID,),
        in_specs=[ispec, ispec],
        out_specs=[ospec, ospec],
        out_shape=(otype, otype),
    )(t1t, t2t)


def _dual_gather(x_grp, table1, table2):
    mesh = plsc.VectorSubcoreMesh(core_axis_name="c", subcore_axis_name="s")

    @functools.partial(
        pl.kernel,
        mesh=mesh,
        compiler_params=pltpu.CompilerParams(use_tc_tiling_on_sc=False),
        out_type=jax.ShapeDtypeStruct((_BF, 2 * _HALF), jnp.float32),
        scratch_types=[
            pltpu.VMEM((_NG, _GRP), jnp.int32),
            pltpu.VMEM((2, _BLK, _HALF), jnp.float32),
            pltpu.VMEM((2, _BLK, _HALF), jnp.float32),
            pltpu.SemaphoreType.DMA,
            pltpu.SemaphoreType.DMA,
        ],
    )
    def k(x_hbm, t1_hbm, t2_hbm, out_hbm, idx_v, rows1_v, rows2_v,
          gsem, wsem):
        wid = lax.axis_index("s") * _NC + lax.axis_index("c")
        base = wid * _PER_W
        pltpu.sync_copy(x_hbm.at[wid], idx_v)

        def outer(i, carry):
            for b in range(2):
                j = 2 * i + b
                ghs = []
                for g in range(_GPB):
                    row = j * _GPB + g
                    ghs.append(pltpu.async_copy(
                        t1_hbm.at[idx_v.at[row]],
                        rows1_v.at[b].at[pl.ds(g * _GRP, _GRP)], gsem))
                    ghs.append(pltpu.async_copy(
                        t2_hbm.at[idx_v.at[row]],
                        rows2_v.at[b].at[pl.ds(g * _GRP, _GRP)], gsem))

                pb = 1 - b
                pbase = base + (j - 1) * _BLK

                @pl.when(j > 0)
                def _fire_writes():
                    pltpu.async_copy(
                        rows1_v.at[pb],
                        out_hbm.at[pl.ds(pbase, _BLK), pl.ds(0, _HALF)],
                        wsem)
                    pltpu.async_copy(
                        rows2_v.at[pb],
                        out_hbm.at[pl.ds(pbase, _BLK), pl.ds(_HALF, _HALF)],
                        wsem)

                for h in ghs:
                    h.wait()

                @pl.when(j > 0)
                def _wait_writes():
                    pltpu.make_async_copy(
                        rows1_v.at[pb],
                        out_hbm.at[pl.ds(pbase, _BLK), pl.ds(0, _HALF)],
                        wsem).wait()
                    pltpu.make_async_copy(
                        rows2_v.at[pb],
                        out_hbm.at[pl.ds(pbase, _BLK), pl.ds(_HALF, _HALF)],
                        wsem).wait()
            return carry

        lax.fori_loop(0, _NBLK // 2, outer, 0)

        lbase = base + (_NBLK - 1) * _BLK
        pltpu.sync_copy(
            rows1_v.at[1], out_hbm.at[pl.ds(lbase, _BLK), pl.ds(0, _HALF)])
        pltpu.sync_copy(
            rows2_v.at[1],
            out_hbm.at[pl.ds(lbase, _BLK), pl.ds(_HALF, _HALF)])

    return k(x_grp, table1, table2)


def kernel(x, table1, table2):
    t1t = jnp.swapaxes(table1, 0, 1)
    t2t = jnp.swapaxes(table2, 0, 1)
    r1, r2 = _relayout_tc(t1t, t2t)
    x_grp = x.reshape(_NW, _NG, _GRP).astype(jnp.int32)
    out = _dual_gather(x_grp, r1.reshape(_NUM_EMB, _HALF),
                       r2.reshape(_NUM_EMB, _HALF))
    return out.reshape(_BATCH, _FIELDS, 2 * _HALF)
